# baseline (speedup 1.0000x reference)
import functools

import jax
import jax.numpy as jnp
from jax import lax
from jax.experimental import pallas as pl
from jax.experimental.pallas import tpu as pltpu
from jax.experimental.pallas import tpu_sc as plsc

_K = 1024
_D = 256
_N = 2 * 224 * 224
_TILE = 1024
_GRID = _N // _TILE

_NC = 2
_NS = 16
_NW = _NC * _NS
_BPW = _N // _NW
_CH = 112
_NCH = _BPW // _CH


def _conv(x, w, b):
    y = lax.conv_general_dilated(
        x, w, window_strides=(1, 1), padding=((1, 1), (1, 1)),
        dimension_numbers=('NCHW', 'OIHW', 'NCHW'))
    return y + b[None, :, None, None]


def _convT(x, w, b):
    w2 = jnp.flip(w, axis=(2, 3)).transpose(1, 0, 2, 3)
    return _conv(x, w2, b)


def _argmin_body(xsq_ref, xf_ref, embT_ref, embsq_ref, idx_ref):
    xf = xf_ref[...]
    dot = jnp.dot(xf, embT_ref[...],
                  preferred_element_type=jnp.float32)
    d2 = (xsq_ref[...] + embsq_ref[...]) - 2.0 * dot
    dist = jnp.sqrt(d2)
    m = jnp.min(dist, axis=1, keepdims=True)
    ids = lax.broadcasted_iota(jnp.int32, dist.shape, 1)
    idx = jnp.min(jnp.where(dist <= m, ids, _K), axis=1)
    idx_ref[0, 0, :] = idx


def _vq_argmin(xsq, xf, embT, embsq):
    out = pl.pallas_call(
        _argmin_body,
        grid=(_GRID,),
        in_specs=[
            pl.BlockSpec((_TILE, 1), lambda i: (i, 0)),
            pl.BlockSpec((_TILE, _D), lambda i: (i, 0)),
            pl.BlockSpec((_D, _K), lambda i: (0, 0)),
            pl.BlockSpec((1, _K), lambda i: (0, 0)),
        ],
        out_specs=pl.BlockSpec((1, 1, _TILE), lambda i: (i, 0, 0)),
        out_shape=jax.ShapeDtypeStruct((_GRID, 1, _TILE), jnp.int32),
        compiler_params=pltpu.CompilerParams(
            dimension_semantics=("arbitrary",)),
    )(xsq, xf, embT, embsq)
    return out.reshape(-1)


@functools.cache
def _make_sc_gather():
    @functools.partial(
        pl.kernel,
        mesh=plsc.VectorSubcoreMesh(core_axis_name="c", subcore_axis_name="s"),
        out_type=jax.ShapeDtypeStruct((_N, _D), jnp.float32),
        scratch_types=[
            pltpu.VMEM((_BPW,), jnp.int32),
            pltpu.VMEM((_CH, _D), jnp.float32),
            pltpu.SemaphoreType.DMA,
        ],
    )
    def _sc_gather(emb_hbm, idx_hbm, out_hbm, idx_v, rows_v, sem):
        wid = lax.axis_index("s") * _NC + lax.axis_index("c")
        base = wid * _BPW
        pltpu.sync_copy(idx_hbm.at[pl.ds(base, _BPW)], idx_v)

        def body(c, carry):
            off = c * _CH
            pltpu.async_copy(emb_hbm.at[idx_v.at[pl.ds(off, _CH)]],
                             rows_v, sem).wait()
            pltpu.sync_copy(rows_v, out_hbm.at[pl.ds(base + off, _CH)])
            return carry

        lax.fori_loop(0, _NCH, body, 0)

    return _sc_gather


def kernel(x, enc_w1, enc_b1, enc_w2, enc_b2, pre_w, pre_b, emb,
           post_w, post_b, dec_w1, dec_b1, dec_w2, dec_b2):
    h = jax.nn.relu(_conv(x, enc_w1, enc_b1))
    h = jax.nn.relu(_conv(h, enc_w2, enc_b2))
    encoded = jax.nn.sigmoid(_conv(h, pre_w, pre_b))

    xp = encoded.transpose(0, 2, 3, 1)
    shp = xp.shape
    xf = xp.reshape(-1, _D)
    xsq = jnp.sum(xf ** 2, axis=1, keepdims=True)
    embsq = jnp.sum(emb ** 2, axis=1)[None, :]

    idx = _vq_argmin(xsq, xf, emb.T, embsq)
    qf = _make_sc_gather()(emb, idx)
    quantized = qf.reshape(shp).transpose(0, 3, 1, 2)

    qsg = encoded + lax.stop_gradient(quantized - encoded)
    h = jax.nn.relu(_convT(qsg, post_w, post_b))
    h = jax.nn.relu(_convT(h, dec_w1, dec_b1))
    recon_x = jax.nn.sigmoid(_convT(h, dec_w2, dec_b2))
    return (encoded, quantized, recon_x)


# speedup vs baseline: 1.0013x; 1.0013x over previous
import functools

import jax
import jax.numpy as jnp
from jax import lax
from jax.experimental import pallas as pl
from jax.experimental.pallas import tpu as pltpu
from jax.experimental.pallas import tpu_sc as plsc

_K = 1024
_D = 256
_N = 2 * 224 * 224
_TILE = 1024
_GRID = _N // _TILE

_NC = 2
_NS = 16
_NW = _NC * _NS
_BPW = _N // _NW
_CH = 112
_NCH = _BPW // _CH


def _conv(x, w, b):
    y = lax.conv_general_dilated(
        x, w, window_strides=(1, 1), padding=((1, 1), (1, 1)),
        dimension_numbers=('NCHW', 'OIHW', 'NCHW'))
    return y + b[None, :, None, None]


def _convT(x, w, b):
    w2 = jnp.flip(w, axis=(2, 3)).transpose(1, 0, 2, 3)
    return _conv(x, w2, b)


def _argmin_body(xsq_ref, xf_ref, embT_ref, embsq_ref, idx_ref):
    xf = xf_ref[...]
    dot = jnp.dot(xf, embT_ref[...],
                  preferred_element_type=jnp.float32)
    d2 = (xsq_ref[...] + embsq_ref[...]) - 2.0 * dot
    dist = jnp.sqrt(d2)
    m = jnp.min(dist, axis=1, keepdims=True)
    ids = lax.broadcasted_iota(jnp.int32, dist.shape, 1)
    idx = jnp.min(jnp.where(dist <= m, ids, _K), axis=1)
    idx_ref[0, 0, :] = idx


def _vq_argmin(xsq, xf, embT, embsq):
    out = pl.pallas_call(
        _argmin_body,
        grid=(_GRID,),
        in_specs=[
            pl.BlockSpec((_TILE, 1), lambda i: (i, 0)),
            pl.BlockSpec((_TILE, _D), lambda i: (i, 0)),
            pl.BlockSpec((_D, _K), lambda i: (0, 0)),
            pl.BlockSpec((1, _K), lambda i: (0, 0)),
        ],
        out_specs=pl.BlockSpec((1, 1, _TILE), lambda i: (i, 0, 0)),
        out_shape=jax.ShapeDtypeStruct((_GRID, 1, _TILE), jnp.int32),
        compiler_params=pltpu.CompilerParams(
            dimension_semantics=("arbitrary",)),
    )(xsq, xf, embT, embsq)
    return out.reshape(-1)


_NBUF = 4


@functools.cache
def _make_sc_gather():
    @functools.partial(
        pl.kernel,
        mesh=plsc.VectorSubcoreMesh(core_axis_name="c", subcore_axis_name="s"),
        out_type=jax.ShapeDtypeStruct((_N, _D), jnp.float32),
        scratch_types=[
            pltpu.VMEM((_BPW,), jnp.int32),
        ] + [pltpu.VMEM((_CH, _D), jnp.float32) for _ in range(_NBUF)]
          + [pltpu.SemaphoreType.DMA for _ in range(_NBUF)],
    )
    def _sc_gather(emb_hbm, idx_hbm, out_hbm, idx_v, *bufs_and_sems):
        bufs = bufs_and_sems[:_NBUF]
        sems = bufs_and_sems[_NBUF:]
        wid = lax.axis_index("s") * _NC + lax.axis_index("c")
        base = wid * _BPW
        pltpu.sync_copy(idx_hbm.at[pl.ds(base, _BPW)], idx_v)

        def start(c, b):
            pltpu.async_copy(emb_hbm.at[idx_v.at[pl.ds(c * _CH, _CH)]],
                             bufs[b], sems[b])

        for b in range(_NBUF - 1):
            start(b, b)

        def body(j, carry):
            for b in range(_NBUF):
                c = _NBUF * j + b

                @pl.when(c + _NBUF - 1 < _NCH)
                def _():
                    start(c + _NBUF - 1, (b + _NBUF - 1) % _NBUF)

                pltpu.make_async_copy(
                    emb_hbm.at[idx_v.at[pl.ds(c * _CH, _CH)]],
                    bufs[b], sems[b]).wait()
                pltpu.sync_copy(bufs[b], out_hbm.at[pl.ds(base + c * _CH, _CH)])
            return carry

        lax.fori_loop(0, _NCH // _NBUF, body, 0)

    return _sc_gather


def kernel(x, enc_w1, enc_b1, enc_w2, enc_b2, pre_w, pre_b, emb,
           post_w, post_b, dec_w1, dec_b1, dec_w2, dec_b2):
    h = jax.nn.relu(_conv(x, enc_w1, enc_b1))
    h = jax.nn.relu(_conv(h, enc_w2, enc_b2))
    encoded = jax.nn.sigmoid(_conv(h, pre_w, pre_b))

    xp = encoded.transpose(0, 2, 3, 1)
    shp = xp.shape
    xf = xp.reshape(-1, _D)
    xsq = jnp.sum(xf ** 2, axis=1, keepdims=True)
    embsq = jnp.sum(emb ** 2, axis=1)[None, :]

    idx = _vq_argmin(xsq, xf, emb.T, embsq)
    qf = _make_sc_gather()(emb, idx)
    quantized = qf.reshape(shp).transpose(0, 3, 1, 2)

    qsg = encoded + lax.stop_gradient(quantized - encoded)
    h = jax.nn.relu(_convT(qsg, post_w, post_b))
    h = jax.nn.relu(_convT(h, dec_w1, dec_b1))
    recon_x = jax.nn.sigmoid(_convT(h, dec_w2, dec_b2))
    return (encoded, quantized, recon_x)


# speedup vs baseline: 2.1023x; 2.0997x over previous
import functools

import jax
import jax.numpy as jnp
from jax import lax
from jax.experimental import pallas as pl
from jax.experimental.pallas import tpu as pltpu
from jax.experimental.pallas import tpu_sc as plsc

_K = 1024
_D = 256
_N = 2 * 224 * 224
_TILE = 1024
_GRID = _N // _TILE

_NC = 2
_NS = 16
_NW = _NC * _NS
_BPW = _N // _NW
_CH = 112
_NCH = _BPW // _CH


def _conv(x, w, b):
    y = lax.conv_general_dilated(
        x, w, window_strides=(1, 1), padding=((1, 1), (1, 1)),
        dimension_numbers=('NCHW', 'OIHW', 'NCHW'))
    return y + b[None, :, None, None]


def _convT(x, w, b):
    w2 = jnp.flip(w, axis=(2, 3)).transpose(1, 0, 2, 3)
    return _conv(x, w2, b)


def _argmin_body(xsq_ref, xf_ref, embT_ref, embsq_ref, idx_ref):
    xf = xf_ref[...]
    dot = jnp.dot(xf, embT_ref[...],
                  preferred_element_type=jnp.float32)
    d2 = (xsq_ref[...] + embsq_ref[...]) - 2.0 * dot
    dist = jnp.sqrt(d2)
    m = jnp.min(dist, axis=1, keepdims=True)
    ids = lax.broadcasted_iota(jnp.int32, dist.shape, 1)
    idx = jnp.min(jnp.where(dist <= m, ids, _K), axis=1)
    idx_ref[0, 0, :] = idx


def _vq_argmin(xsq, xf, embT, embsq):
    out = pl.pallas_call(
        _argmin_body,
        grid=(_GRID,),
        in_specs=[
            pl.BlockSpec((_TILE, 1), lambda i: (i, 0)),
            pl.BlockSpec((_TILE, _D), lambda i: (i, 0)),
            pl.BlockSpec((_D, _K), lambda i: (0, 0)),
            pl.BlockSpec((1, _K), lambda i: (0, 0)),
        ],
        out_specs=pl.BlockSpec((1, 1, _TILE), lambda i: (i, 0, 0)),
        out_shape=jax.ShapeDtypeStruct((_GRID, 1, _TILE), jnp.int32),
        compiler_params=pltpu.CompilerParams(
            dimension_semantics=("arbitrary",)),
    )(xsq, xf, embT, embsq)
    return out.reshape(-1)


_NBUF = 4
_REP = 32


@functools.cache
def _make_sc_gather():
    @functools.partial(
        pl.kernel,
        mesh=plsc.VectorSubcoreMesh(core_axis_name="c", subcore_axis_name="s"),
        out_type=jax.ShapeDtypeStruct((_N, _D), jnp.float32),
        scratch_types=[
            pltpu.VMEM((_BPW,), jnp.int32),
        ] + [pltpu.VMEM((_CH, _D), jnp.float32) for _ in range(_NBUF)]
          + [pltpu.SemaphoreType.DMA for _ in range(_NBUF)],
    )
    def _sc_gather(emb_hbm, idx_hbm, out_hbm, idx_v, *bufs_and_sems):
        bufs = bufs_and_sems[:_NBUF]
        sems = bufs_and_sems[_NBUF:]
        wid = lax.axis_index("s") * _NC + lax.axis_index("c")
        base = wid * _BPW
        pltpu.sync_copy(idx_hbm.at[pl.ds(base, _BPW)], idx_v)

        def start(c, b):
            pltpu.async_copy(emb_hbm.at[idx_v.at[pl.ds(c * _CH, _CH)]],
                             bufs[b], sems[b])

        for b in range(_NBUF - 1):
            start(b, b)

        def body(j, carry):
            for b in range(_NBUF):
                c = _NBUF * j + b

                @pl.when(c + _NBUF - 1 < _NCH)
                def _():
                    start(c + _NBUF - 1, (b + _NBUF - 1) % _NBUF)

                pltpu.make_async_copy(
                    emb_hbm.at[idx_v.at[pl.ds(c * _CH, _CH)]],
                    bufs[b], sems[b]).wait()
                pltpu.sync_copy(bufs[b], out_hbm.at[pl.ds(base + c * _CH, _CH)])
            return carry

        lax.fori_loop(0, _NCH // _NBUF, body, 0)

    return _sc_gather


def kernel(x, enc_w1, enc_b1, enc_w2, enc_b2, pre_w, pre_b, emb,
           post_w, post_b, dec_w1, dec_b1, dec_w2, dec_b2):
    h = jax.nn.relu(_conv(x, enc_w1, enc_b1))
    h = jax.nn.relu(_conv(h, enc_w2, enc_b2))
    encoded = jax.nn.sigmoid(_conv(h, pre_w, pre_b))

    xp = encoded.transpose(0, 2, 3, 1)
    shp = xp.shape
    xf = xp.reshape(-1, _D)
    xsq = jnp.sum(xf ** 2, axis=1, keepdims=True)
    embsq = jnp.sum(emb ** 2, axis=1)[None, :]

    idx = _vq_argmin(xsq, xf, emb.T, embsq)
    emb_rep = jnp.tile(emb, (_REP, 1))
    rep_off = (jnp.arange(_N, dtype=jnp.int32) // _BPW % _REP) * _K
    qf = _make_sc_gather()(emb_rep, idx + rep_off)
    quantized = qf.reshape(shp).transpose(0, 3, 1, 2)

    qsg = encoded + lax.stop_gradient(quantized - encoded)
    h = jax.nn.relu(_convT(qsg, post_w, post_b))
    h = jax.nn.relu(_convT(h, dec_w1, dec_b1))
    recon_x = jax.nn.sigmoid(_convT(h, dec_w2, dec_b2))
    return (encoded, quantized, recon_x)


# speedup vs baseline: 3.6624x; 1.7420x over previous
import functools

import jax
import jax.numpy as jnp
from jax import lax
from jax.experimental import pallas as pl
from jax.experimental.pallas import tpu as pltpu
from jax.experimental.pallas import tpu_sc as plsc

_K = 1024
_D = 256
_N = 2 * 224 * 224
_TILE = 1024
_GRID = _N // _TILE

_NC = 2
_NS = 16
_NW = _NC * _NS
_BPW = _N // _NW
_CH = 112
_NCH = _BPW // _CH


def _conv(x, w, b):
    y = lax.conv_general_dilated(
        x, w, window_strides=(1, 1), padding=((1, 1), (1, 1)),
        dimension_numbers=('NCHW', 'OIHW', 'NCHW'))
    return y + b[None, :, None, None]


def _convT(x, w, b):
    w2 = jnp.flip(w, axis=(2, 3)).transpose(1, 0, 2, 3)
    return _conv(x, w2, b)


def _argmin_body(xsq_ref, xf_ref, embT_ref, embsq_ref, idx_ref):
    xf = xf_ref[...]
    dot = jnp.dot(xf, embT_ref[...],
                  preferred_element_type=jnp.float32)
    d2 = (xsq_ref[...] + embsq_ref[...]) - 2.0 * dot
    dist = jnp.sqrt(d2)
    m = jnp.min(dist, axis=1, keepdims=True)
    ids = lax.broadcasted_iota(jnp.int32, dist.shape, 1)
    idx = jnp.min(jnp.where(dist <= m, ids, _K), axis=1)
    idx_ref[0, 0, :] = idx


def _vq_argmin(xsq, xf, embT, embsq):
    out = pl.pallas_call(
        _argmin_body,
        grid=(_GRID,),
        in_specs=[
            pl.BlockSpec((_TILE, 1), lambda i: (i, 0)),
            pl.BlockSpec((_TILE, _D), lambda i: (i, 0)),
            pl.BlockSpec((_D, _K), lambda i: (0, 0)),
            pl.BlockSpec((1, _K), lambda i: (0, 0)),
        ],
        out_specs=pl.BlockSpec((1, 1, _TILE), lambda i: (i, 0, 0)),
        out_shape=jax.ShapeDtypeStruct((_GRID, 1, _TILE), jnp.int32),
        compiler_params=pltpu.CompilerParams(
            dimension_semantics=("arbitrary",)),
    )(xsq, xf, embT, embsq)
    return out.reshape(-1)


_NBUF = 4
_REP = 32


@functools.cache
def _make_sc_gather():
    @functools.partial(
        pl.kernel,
        mesh=plsc.VectorSubcoreMesh(core_axis_name="c", subcore_axis_name="s"),
        out_type=jax.ShapeDtypeStruct((_N, _D), jnp.float32),
        scratch_types=[
            pltpu.VMEM((_BPW,), jnp.int32),
        ] + [pltpu.VMEM((_CH, _D), jnp.float32) for _ in range(_NBUF)]
          + [pltpu.SemaphoreType.DMA for _ in range(_NBUF)],
    )
    def _sc_gather(emb_hbm, idx_hbm, out_hbm, idx_v, *bufs_and_sems):
        bufs = bufs_and_sems[:_NBUF]
        sems = bufs_and_sems[_NBUF:]
        wid = lax.axis_index("s") * _NC + lax.axis_index("c")
        base = wid * _BPW
        pltpu.sync_copy(idx_hbm.at[pl.ds(base, _BPW)], idx_v)

        def start(c, b):
            pltpu.async_copy(emb_hbm.at[idx_v.at[pl.ds(c * _CH, _CH)]],
                             bufs[b], sems[b])

        for b in range(_NBUF - 1):
            start(b, b)

        def body(j, carry):
            for b in range(_NBUF):
                c = _NBUF * j + b

                @pl.when(c + _NBUF - 1 < _NCH)
                def _():
                    start(c + _NBUF - 1, (b + _NBUF - 1) % _NBUF)

                pltpu.make_async_copy(
                    emb_hbm.at[idx_v.at[pl.ds(c * _CH, _CH)]],
                    bufs[b], sems[b]).wait()
                pltpu.sync_copy(bufs[b], out_hbm.at[pl.ds(base + c * _CH, _CH)])
            return carry

        lax.fori_loop(0, _NCH // _NBUF, body, 0)

    return _sc_gather


_TSZ = 1032
_PPW = _N // _NW
_PHALF = _PPW // 2
_VPH = _PHALF // 16


@functools.cache
def _make_sc_postconv():
    @functools.partial(
        pl.kernel,
        mesh=plsc.VectorSubcoreMesh(core_axis_name="c", subcore_axis_name="s"),
        out_type=jax.ShapeDtypeStruct((2 * 8 * _S,), jnp.float32),
        scratch_types=[
            pltpu.VMEM((8 * 9 * _TSZ,), jnp.float32),
            pltpu.VMEM((9, _PHALF), jnp.int32),
            pltpu.VMEM((8 * _PHALF,), jnp.float32),
            pltpu.VMEM((8, 16), jnp.float32),
        ],
        compiler_params=pltpu.CompilerParams(needs_layout_passes=False),
    )
    def _sc_postconv(tab_hbm, taps_hbm, bias_hbm, out_hbm,
                     tab_v, taps_v, out_v, bias_v):
        wid = lax.axis_index("s") * _NC + lax.axis_index("c")
        img = wid // 16
        pltpu.sync_copy(tab_hbm, tab_v)
        pltpu.sync_copy(bias_hbm, bias_v)

        for half in range(2):
            pltpu.sync_copy(taps_hbm.at[wid, half], taps_v)

            def body(i, carry):
                vs = [taps_v[t, pl.ds(i * 16, 16)] for t in range(9)]
                for co in range(8):
                    acc = bias_v[co, :]
                    for t in range(9):
                        acc = acc + plsc.load_gather(
                            tab_v, [vs[t] + (co * 9 + t) * _TSZ])
                    out_v[pl.ds(co * _PHALF + i * 16, 16)] = (
                        jnp.maximum(acc, 0.0))
                return carry

            lax.fori_loop(0, _VPH, body, 0)
            s0 = (wid % 16) * _PPW + half * _PHALF
            for co in range(8):
                pltpu.sync_copy(
                    out_v.at[pl.ds(co * _PHALF, _PHALF)],
                    out_hbm.at[pl.ds((img * 8 + co) * _S + s0, _PHALF)])

    return _sc_postconv


_S = 50176


_CHS = 3584
_NCHS = _S // _CHS


def _dec_body(x_ref, w1_ref, b1_ref, w2_ref, b2_ref, out_ref):
    X = x_ref[0]
    pxm = lax.broadcasted_iota(jnp.int32, (1, _S), 1)
    w_of = pxm % 224
    masks = []
    for dh in (-1, 0, 1):
        for dw in (-1, 0, 1):
            cond = (pxm >= 0)
            if dh == -1:
                cond = cond & (pxm >= 224)
            elif dh == 1:
                cond = cond & (pxm < _S - 224)
            if dw == -1:
                cond = cond & (w_of >= 1)
            elif dw == 1:
                cond = cond & (w_of <= 222)
            masks.append(jnp.where(cond, 1.0, 0.0))

    def conv_full(Xin, w_ref, cout, bias):
        acc = jnp.zeros((cout, _S), jnp.float32) + bias
        k = 0
        for dh in (-1, 0, 1):
            for dw in (-1, 0, 1):
                shift = dh * 224 + dw
                xs = pltpu.roll(Xin, (-shift) % _S, 1) if shift else Xin
                term = jnp.dot(w_ref[k], xs,
                               preferred_element_type=jnp.float32)
                acc = acc + term * masks[k]
                k += 1
        return acc

    y1 = jnp.maximum(conv_full(X, w1_ref, 16, b1_ref[...]), 0.0)
    y2 = conv_full(y1, w2_ref, 8, b2_ref[...])
    out_ref[0] = jax.nn.sigmoid(y2)[0:1, :]


def _dec_tc(h1, w1t, b1, w2p, b2p):
    return pl.pallas_call(
        _dec_body,
        grid=(2,),
        in_specs=[
            pl.BlockSpec((1, 8, _S), lambda n: (n, 0, 0)),
            pl.BlockSpec((9, 16, 8), lambda n: (0, 0, 0)),
            pl.BlockSpec((16, 1), lambda n: (0, 0)),
            pl.BlockSpec((9, 8, 16), lambda n: (0, 0, 0)),
            pl.BlockSpec((8, 1), lambda n: (0, 0)),
        ],
        out_specs=pl.BlockSpec((1, 1, _S), lambda n: (n, 0, 0)),
        out_shape=jax.ShapeDtypeStruct((2, 1, _S), jnp.float32),
        compiler_params=pltpu.CompilerParams(
            dimension_semantics=("arbitrary",)),
    )(h1, w1t, b1, w2p, b2p)


def kernel(x, enc_w1, enc_b1, enc_w2, enc_b2, pre_w, pre_b, emb,
           post_w, post_b, dec_w1, dec_b1, dec_w2, dec_b2):
    h = jax.nn.relu(_conv(x, enc_w1, enc_b1))
    h = jax.nn.relu(_conv(h, enc_w2, enc_b2))
    encoded = jax.nn.sigmoid(_conv(h, pre_w, pre_b))

    xp = encoded.transpose(0, 2, 3, 1)
    shp = xp.shape
    xf = xp.reshape(-1, _D)
    xsq = jnp.sum(xf ** 2, axis=1, keepdims=True)
    embsq = jnp.sum(emb ** 2, axis=1)[None, :]

    idx = _vq_argmin(xsq, xf, emb.T, embsq)

    emb_rep = jnp.tile(emb, (_REP, 1))
    rep_off = (jnp.arange(_N, dtype=jnp.int32) // _BPW % _REP) * _K
    qf = _make_sc_gather()(emb_rep, idx + rep_off)
    quantized = qf.reshape(shp).transpose(0, 3, 1, 2)

    w2 = jnp.flip(post_w, axis=(2, 3)).transpose(1, 0, 2, 3)
    w2r = w2.reshape(8, _D, 9)
    T = jnp.einsum('ic,ock->oki', emb, w2r,
                   precision=lax.Precision.HIGHEST)
    tab = jnp.zeros((8, 9, _TSZ), jnp.float32).at[:, :, :_K].set(T)
    bias16 = jnp.tile(post_b[:, None], (1, 16))

    idx_img = idx.reshape(2, 224, 224)
    padi = jnp.pad(idx_img, ((0, 0), (1, 1), (1, 1)), constant_values=_K)
    taps = jnp.stack([padi[:, dh:dh + 224, dw:dw + 224].reshape(-1)
                      for dh in range(3) for dw in range(3)])
    tapsw = taps.reshape(9, _NW, 2, _PHALF).transpose(1, 2, 0, 3)

    h1 = _make_sc_postconv()(tab.reshape(-1), tapsw,
                             bias16).reshape(2, 8, _S)

    e1 = jnp.flip(dec_w1, axis=(2, 3)).transpose(1, 0, 2, 3)
    w1t = e1.transpose(2, 3, 0, 1).reshape(9, 16, 8)
    e2 = jnp.flip(dec_w2, axis=(2, 3)).transpose(1, 0, 2, 3)
    w2t = e2.transpose(2, 3, 0, 1).reshape(9, 1, 16)
    w2p = jnp.zeros((9, 8, 16), jnp.float32).at[:, 0:1, :].set(w2t)
    b2p = jnp.zeros((8, 1), jnp.float32).at[0, 0].set(dec_b2[0])

    recon = _dec_tc(h1, w1t, dec_b1[:, None], w2p, b2p)
    recon_x = recon.reshape(2, 1, 224, 224)
    return (encoded, quantized, recon_x)


# speedup vs baseline: 4.5030x; 1.2295x over previous
import functools

import jax
import jax.numpy as jnp
from jax import lax
from jax.experimental import pallas as pl
from jax.experimental.pallas import tpu as pltpu
from jax.experimental.pallas import tpu_sc as plsc

_K = 1024
_D = 256
_N = 2 * 224 * 224
_TILE = 1024
_GRID = _N // _TILE

_NC = 2
_NS = 16
_NW = _NC * _NS
_BPW = _N // _NW
_CH = 112
_NCH = _BPW // _CH


def _conv(x, w, b):
    y = lax.conv_general_dilated(
        x, w, window_strides=(1, 1), padding=((1, 1), (1, 1)),
        dimension_numbers=('NCHW', 'OIHW', 'NCHW'))
    return y + b[None, :, None, None]


def _convT(x, w, b):
    w2 = jnp.flip(w, axis=(2, 3)).transpose(1, 0, 2, 3)
    return _conv(x, w2, b)


def _argmin_body(xsq_ref, xf_ref, embT_ref, embsq_ref, idx_ref):
    xf = xf_ref[...]
    dot = jnp.dot(xf, embT_ref[...],
                  preferred_element_type=jnp.float32)
    d2 = (xsq_ref[...] + embsq_ref[...]) - 2.0 * dot
    dist = jnp.sqrt(d2)
    m = jnp.min(dist, axis=1, keepdims=True)
    ids = lax.broadcasted_iota(jnp.int32, dist.shape, 1)
    idx = jnp.min(jnp.where(dist <= m, ids, _K), axis=1)
    idx_ref[0, 0, :] = idx


def _vq_argmin(xsq, xf, embT, embsq):
    out = pl.pallas_call(
        _argmin_body,
        grid=(_GRID,),
        in_specs=[
            pl.BlockSpec((_TILE, 1), lambda i: (i, 0)),
            pl.BlockSpec((_TILE, _D), lambda i: (i, 0)),
            pl.BlockSpec((_D, _K), lambda i: (0, 0)),
            pl.BlockSpec((1, _K), lambda i: (0, 0)),
        ],
        out_specs=pl.BlockSpec((1, 1, _TILE), lambda i: (i, 0, 0)),
        out_shape=jax.ShapeDtypeStruct((_GRID, 1, _TILE), jnp.int32),
        compiler_params=pltpu.CompilerParams(
            dimension_semantics=("arbitrary",)),
    )(xsq, xf, embT, embsq)
    return out.reshape(-1)


_NBUF = 4
_REP = 32


@functools.cache
def _make_sc_gather():
    @functools.partial(
        pl.kernel,
        mesh=plsc.VectorSubcoreMesh(core_axis_name="c", subcore_axis_name="s"),
        out_type=jax.ShapeDtypeStruct((_N, _D), jnp.float32),
        scratch_types=[
            pltpu.VMEM((_BPW,), jnp.int32),
        ] + [pltpu.VMEM((_CH, _D), jnp.float32) for _ in range(_NBUF)]
          + [pltpu.SemaphoreType.DMA for _ in range(_NBUF)],
    )
    def _sc_gather(emb_hbm, idx_hbm, out_hbm, idx_v, *bufs_and_sems):
        bufs = bufs_and_sems[:_NBUF]
        sems = bufs_and_sems[_NBUF:]
        wid = lax.axis_index("s") * _NC + lax.axis_index("c")
        base = wid * _BPW
        pltpu.sync_copy(idx_hbm.at[pl.ds(base, _BPW)], idx_v)

        def start(c, b):
            pltpu.async_copy(emb_hbm.at[idx_v.at[pl.ds(c * _CH, _CH)]],
                             bufs[b], sems[b])

        for b in range(_NBUF - 1):
            start(b, b)

        def body(j, carry):
            for b in range(_NBUF):
                c = _NBUF * j + b

                @pl.when(c + _NBUF - 1 < _NCH)
                def _():
                    start(c + _NBUF - 1, (b + _NBUF - 1) % _NBUF)

                pltpu.make_async_copy(
                    emb_hbm.at[idx_v.at[pl.ds(c * _CH, _CH)]],
                    bufs[b], sems[b]).wait()
                pltpu.sync_copy(bufs[b], out_hbm.at[pl.ds(base + c * _CH, _CH)])
            return carry

        lax.fori_loop(0, _NCH // _NBUF, body, 0)

    return _sc_gather


_TSZ = 1032
_PPW = _N // _NW
_PHALF = _PPW // 2
_VPH = _PHALF // 16


@functools.cache
def _make_sc_postconv():
    @functools.partial(
        pl.kernel,
        mesh=plsc.VectorSubcoreMesh(core_axis_name="c", subcore_axis_name="s"),
        out_type=jax.ShapeDtypeStruct((2 * 8 * _S,), jnp.float32),
        scratch_types=[
            pltpu.VMEM((8 * 9 * _TSZ,), jnp.float32),
            pltpu.VMEM((9, _PHALF), jnp.int32),
            pltpu.VMEM((8 * _PHALF,), jnp.float32),
            pltpu.VMEM((8, 16), jnp.float32),
        ],
        compiler_params=pltpu.CompilerParams(needs_layout_passes=False),
    )
    def _sc_postconv(tab_hbm, taps_hbm, bias_hbm, out_hbm,
                     tab_v, taps_v, out_v, bias_v):
        wid = lax.axis_index("s") * _NC + lax.axis_index("c")
        img = wid // 16
        pltpu.sync_copy(tab_hbm, tab_v)
        pltpu.sync_copy(bias_hbm, bias_v)

        for half in range(2):
            pltpu.sync_copy(taps_hbm.at[wid, half], taps_v)

            def body(i, carry):
                vs = [taps_v[t, pl.ds(i * 16, 16)] for t in range(9)]
                for co in range(8):
                    acc = bias_v[co, :]
                    for t in range(9):
                        acc = acc + plsc.load_gather(
                            tab_v, [vs[t] + (co * 9 + t) * _TSZ])
                    out_v[pl.ds(co * _PHALF + i * 16, 16)] = (
                        jnp.maximum(acc, 0.0))
                return carry

            lax.fori_loop(0, _VPH, body, 0)
            s0 = (wid % 16) * _PPW + half * _PHALF
            for co in range(8):
                pltpu.sync_copy(
                    out_v.at[pl.ds(co * _PHALF, _PHALF)],
                    out_hbm.at[pl.ds((img * 8 + co) * _S + s0, _PHALF)])

    return _sc_postconv


_S = 50176


_CHS = 3584
_NCHS = _S // _CHS


def _dec_body(x_ref, w1_ref, b1_ref, w2_ref, b2_ref, out_ref):
    X = x_ref[0]
    pxm = lax.broadcasted_iota(jnp.int32, (1, _S), 1)
    w_of = pxm % 224
    masks = []
    for dh in (-1, 0, 1):
        for dw in (-1, 0, 1):
            cond = (pxm >= 0)
            if dh == -1:
                cond = cond & (pxm >= 224)
            elif dh == 1:
                cond = cond & (pxm < _S - 224)
            if dw == -1:
                cond = cond & (w_of >= 1)
            elif dw == 1:
                cond = cond & (w_of <= 222)
            masks.append(jnp.where(cond, 1.0, 0.0))

    def conv_full(Xin, w_ref, cout, bias):
        acc = jnp.zeros((cout, _S), jnp.float32) + bias
        k = 0
        for dh in (-1, 0, 1):
            for dw in (-1, 0, 1):
                shift = dh * 224 + dw
                xs = pltpu.roll(Xin, (-shift) % _S, 1) if shift else Xin
                term = jnp.dot(w_ref[k], xs,
                               preferred_element_type=jnp.float32)
                acc = acc + term * masks[k]
                k += 1
        return acc

    y1 = jnp.maximum(conv_full(X, w1_ref, 16, b1_ref[...]), 0.0)
    y2 = conv_full(y1, w2_ref, 8, b2_ref[...])
    out_ref[0] = jax.nn.sigmoid(y2)[0:1, :]


def _dec_tc(h1, w1t, b1, w2p, b2p):
    return pl.pallas_call(
        _dec_body,
        grid=(2,),
        in_specs=[
            pl.BlockSpec((1, 8, _S), lambda n: (n, 0, 0)),
            pl.BlockSpec((9, 16, 8), lambda n: (0, 0, 0)),
            pl.BlockSpec((16, 1), lambda n: (0, 0)),
            pl.BlockSpec((9, 8, 16), lambda n: (0, 0, 0)),
            pl.BlockSpec((8, 1), lambda n: (0, 0)),
        ],
        out_specs=pl.BlockSpec((1, 1, _S), lambda n: (n, 0, 0)),
        out_shape=jax.ShapeDtypeStruct((2, 1, _S), jnp.float32),
        compiler_params=pltpu.CompilerParams(
            dimension_semantics=("arbitrary",)),
    )(h1, w1t, b1, w2p, b2p)


def _conv_nhwc(x, w, b):
    y = lax.conv_general_dilated(
        x, w.transpose(2, 3, 1, 0), window_strides=(1, 1),
        padding=((1, 1), (1, 1)),
        dimension_numbers=('NHWC', 'HWIO', 'NHWC'))
    return y + b[None, None, None, :]


def kernel(x, enc_w1, enc_b1, enc_w2, enc_b2, pre_w, pre_b, emb,
           post_w, post_b, dec_w1, dec_b1, dec_w2, dec_b2):
    xt = x.transpose(0, 2, 3, 1)
    h = jax.nn.relu(_conv_nhwc(xt, enc_w1, enc_b1))
    h = jax.nn.relu(_conv_nhwc(h, enc_w2, enc_b2))
    enc_nhwc = jax.nn.sigmoid(_conv_nhwc(h, pre_w, pre_b))
    encoded = enc_nhwc.transpose(0, 3, 1, 2)

    shp = enc_nhwc.shape
    xf = enc_nhwc.reshape(-1, _D)
    xsq = jnp.sum(xf ** 2, axis=1, keepdims=True)
    embsq = jnp.sum(emb ** 2, axis=1)[None, :]

    idx = _vq_argmin(xsq, xf, emb.T, embsq)

    emb_rep = jnp.tile(emb, (_REP, 1))
    rep_off = (jnp.arange(_N, dtype=jnp.int32) // _BPW % _REP) * _K
    qf = _make_sc_gather()(emb_rep, idx + rep_off)
    quantized = qf.reshape(shp).transpose(0, 3, 1, 2)

    w2 = jnp.flip(post_w, axis=(2, 3)).transpose(1, 0, 2, 3)
    w2r = w2.reshape(8, _D, 9)
    T = jnp.einsum('ic,ock->oki', emb, w2r,
                   precision=lax.Precision.HIGHEST)
    tab = jnp.zeros((8, 9, _TSZ), jnp.float32).at[:, :, :_K].set(T)
    bias16 = jnp.tile(post_b[:, None], (1, 16))

    idx_img = idx.reshape(2, 224, 224)
    padi = jnp.pad(idx_img, ((0, 0), (1, 1), (1, 1)), constant_values=_K)
    taps = jnp.stack([padi[:, dh:dh + 224, dw:dw + 224].reshape(-1)
                      for dh in range(3) for dw in range(3)])
    tapsw = taps.reshape(9, _NW, 2, _PHALF).transpose(1, 2, 0, 3)

    h1 = _make_sc_postconv()(tab.reshape(-1), tapsw,
                             bias16).reshape(2, 8, _S)

    e1 = jnp.flip(dec_w1, axis=(2, 3)).transpose(1, 0, 2, 3)
    w1t = e1.transpose(2, 3, 0, 1).reshape(9, 16, 8)
    e2 = jnp.flip(dec_w2, axis=(2, 3)).transpose(1, 0, 2, 3)
    w2t = e2.transpose(2, 3, 0, 1).reshape(9, 1, 16)
    w2p = jnp.zeros((9, 8, 16), jnp.float32).at[:, 0:1, :].set(w2t)
    b2p = jnp.zeros((8, 1), jnp.float32).at[0, 0].set(dec_b2[0])

    recon = _dec_tc(h1, w1t, dec_b1[:, None], w2p, b2p)
    recon_x = recon.reshape(2, 1, 224, 224)
    return (encoded, quantized, recon_x)
